# baseline (speedup 1.0000x reference)
import jax
import jax.numpy as jnp
from jax import lax
from jax.experimental import pallas as pl
from jax.experimental.pallas import tpu as pltpu
from jax.experimental.pallas import tpu_sc as plsc

_NUM_CORES = 1
_NUM_SUBCORES = 16
_NUM_WORKERS = _NUM_CORES * _NUM_SUBCORES
_CHUNK_ROWS = 32
_NBUF = 3


def _make_sc_lookup(rows_per_worker, chunk_rows):
    n_chunks = rows_per_worker // chunk_rows
    assert n_chunks >= _NBUF

    def _sc_lookup(table_hbm, out_hbm, buf, in_sem, out_sem):
        wid = lax.axis_index("s") * _NUM_CORES + lax.axis_index("c")
        base = wid * rows_per_worker

        def in_copy(i):
            return pltpu.make_async_copy(
                table_hbm.at[pl.ds(base + i * chunk_rows, chunk_rows)],
                buf.at[i % _NBUF], in_sem)

        def out_copy(i):
            return pltpu.make_async_copy(
                buf.at[i % _NBUF],
                out_hbm.at[pl.ds(base + i * chunk_rows, chunk_rows)], out_sem)

        for j in range(_NBUF):
            in_copy(j).start()
        for i in range(n_chunks):
            if i >= 1 and i + _NBUF - 1 < n_chunks:
                out_copy(i - 1).wait()
                in_copy(i + _NBUF - 1).start()
            in_copy(i).wait()
            out_copy(i).start()
        for i in range(n_chunks - _NBUF, n_chunks):
            out_copy(i).wait()

    return _sc_lookup


def kernel(input_ids, pos_table):
    seq_len = input_ids.shape[1]
    hidden = pos_table.shape[1]
    assert seq_len % (_NUM_WORKERS * _CHUNK_ROWS) == 0
    rows_per_worker = seq_len // _NUM_WORKERS
    mesh = plsc.VectorSubcoreMesh(core_axis_name="c", subcore_axis_name="s",
                                  num_cores=1)
    out = pl.kernel(
        _make_sc_lookup(rows_per_worker, _CHUNK_ROWS),
        out_type=jax.ShapeDtypeStruct((seq_len, hidden), pos_table.dtype),
        mesh=mesh,
        scratch_types=[
            pltpu.VMEM((_NBUF, _CHUNK_ROWS, hidden), pos_table.dtype),
            pltpu.SemaphoreType.DMA,
            pltpu.SemaphoreType.DMA,
        ],
    )(pos_table)
    return out[None]


# speedup vs baseline: 1.0545x; 1.0545x over previous
import jax
import jax.numpy as jnp
from jax import lax
from jax.experimental import pallas as pl
from jax.experimental.pallas import tpu as pltpu
from jax.experimental.pallas import tpu_sc as plsc

_NUM_CORES = 2
_NUM_SUBCORES = 16
_NUM_WORKERS = _NUM_CORES * _NUM_SUBCORES
_CHUNK_ROWS = 16
_NBUF = 6
_LAG = 3


def _make_sc_lookup(rows_per_worker, chunk_rows):
    n_chunks = rows_per_worker // chunk_rows
    assert n_chunks >= _NBUF + 1

    def _sc_lookup(table_hbm, out_hbm, buf, in_sem, out_sem):
        wid = lax.axis_index("s") * _NUM_CORES + lax.axis_index("c")
        base = wid * rows_per_worker

        def in_copy(i):
            return pltpu.make_async_copy(
                table_hbm.at[pl.ds(base + i * chunk_rows, chunk_rows)],
                buf.at[i % _NBUF], in_sem)

        def out_copy(i):
            return pltpu.make_async_copy(
                buf.at[i % _NBUF],
                out_hbm.at[pl.ds(base + i * chunk_rows, chunk_rows)], out_sem)

        for j in range(_LAG):
            in_copy(j).start()
        for i in range(n_chunks):
            in_copy(i).wait()
            out_copy(i).start()
            j = i + _LAG
            if j < n_chunks:
                if j >= _NBUF:
                    out_copy(j - _NBUF).wait()
                in_copy(j).start()
        for i in range(n_chunks - _NBUF, n_chunks):
            out_copy(i).wait()

    return _sc_lookup


def kernel(input_ids, pos_table):
    seq_len = input_ids.shape[1]
    hidden = pos_table.shape[1]
    assert seq_len % (_NUM_WORKERS * _CHUNK_ROWS) == 0
    rows_per_worker = seq_len // _NUM_WORKERS
    mesh = plsc.VectorSubcoreMesh(core_axis_name="c", subcore_axis_name="s")
    out = pl.kernel(
        _make_sc_lookup(rows_per_worker, _CHUNK_ROWS),
        out_type=jax.ShapeDtypeStruct((seq_len, hidden), pos_table.dtype),
        mesh=mesh,
        scratch_types=[
            pltpu.VMEM((_NBUF, _CHUNK_ROWS, hidden), pos_table.dtype),
            pltpu.SemaphoreType.DMA,
            pltpu.SemaphoreType.DMA,
        ],
    )(pos_table)
    return out[None]


# speedup vs baseline: 1.0614x; 1.0065x over previous
import jax
from jax import lax
from jax.experimental import pallas as pl
from jax.experimental.pallas import tpu as pltpu
from jax.experimental.pallas import tpu_sc as plsc

_NUM_CORES = 2
_NUM_SUBCORES = 16
_NUM_WORKERS = _NUM_CORES * _NUM_SUBCORES
_CHUNK_ROWS = 16
_NBUF = 6
_LAG = 3


def _make_sc_lookup(rows_per_worker, chunk_rows):
    n_chunks = rows_per_worker // chunk_rows
    assert n_chunks >= _NBUF + 1

    def _sc_lookup(table_hbm, out_hbm, buf, in_sem, out_sem):
        wid = lax.axis_index("s") * _NUM_CORES + lax.axis_index("c")
        base = wid * rows_per_worker

        def in_copy(i):
            return pltpu.make_async_copy(
                table_hbm.at[pl.ds(base + i * chunk_rows, chunk_rows)],
                buf.at[i % _NBUF], in_sem)

        def out_copy(i):
            return pltpu.make_async_copy(
                buf.at[i % _NBUF],
                out_hbm.at[pl.ds(base + i * chunk_rows, chunk_rows)], out_sem)

        for j in range(_LAG):
            in_copy(j).start()
        for i in range(n_chunks):
            in_copy(i).wait()
            out_copy(i).start()
            j = i + _LAG
            if j < n_chunks:
                if j >= _NBUF:
                    out_copy(j - _NBUF).wait()
                in_copy(j).start()
        for i in range(n_chunks - _NBUF, n_chunks):
            out_copy(i).wait()

    return _sc_lookup


def kernel(input_ids, pos_table):
    seq_len = input_ids.shape[1]
    hidden = pos_table.shape[1]
    assert seq_len % (_NUM_WORKERS * _CHUNK_ROWS) == 0
    rows_per_worker = seq_len // _NUM_WORKERS
    mesh = plsc.VectorSubcoreMesh(core_axis_name="c", subcore_axis_name="s")
    out = pl.kernel(
        _make_sc_lookup(rows_per_worker, _CHUNK_ROWS),
        out_type=jax.ShapeDtypeStruct((seq_len, hidden), pos_table.dtype),
        mesh=mesh,
        scratch_types=[
            pltpu.VMEM((_NBUF, _CHUNK_ROWS, hidden), pos_table.dtype),
            pltpu.SemaphoreType.DMA,
            pltpu.SemaphoreType.DMA,
        ],
    )(pos_table)
    return out[None]
